# baseline (speedup 1.0000x reference)
import functools

import jax
import jax.numpy as jnp
from jax import lax
from jax.experimental import pallas as pl
from jax.experimental.pallas import tpu as pltpu
from jax.experimental.pallas import tpu_sc as plsc

N = 10000
E = 160000
D = 256
Q = 64
NQ = 4
NC = 2
NS = 16
L = 16
EPW = E // NS
CHUNK = 80
NCHUNK = EPW // CHUNK
NP = 10000
RPW = NP // NS
ZR = 125
NB = 5


def _sc_agg(weighted: bool):
    mesh = plsc.VectorSubcoreMesh(core_axis_name="c", subcore_axis_name="s",
                                  num_cores=NC, num_subcores=NS)

    scratch = [
        pltpu.VMEM((NCHUNK, CHUNK), jnp.int32),
        pltpu.VMEM((NCHUNK, CHUNK), jnp.int32),
        *[pltpu.VMEM((CHUNK, Q), jnp.float32) for _ in range(NB)],
        pltpu.VMEM((ZR, Q), jnp.float32),
        pltpu.VMEM_SHARED((NP, Q), jnp.float32),
        *[pltpu.SemaphoreType.DMA for _ in range(2 * NB)],
    ]
    if weighted:
        scratch.insert(2, pltpu.VMEM((NCHUNK, CHUNK), jnp.float32))

    def body(h_hbm, src_hbm, dst_hbm, *rest):
        if weighted:
            ew_hbm, out_hbm, src_v, dst_v, ew_v = rest[:5]
            rest = rest[5:]
        else:
            out_hbm, src_v, dst_v = rest[:3]
            rest = rest[3:]
        rows = rest[:NB]
        zbuf, acc_sh = rest[NB], rest[NB + 1]
        gsem = rest[NB + 2:NB + 2 + NB]
        ssem = rest[NB + 2 + NB:]
        c = lax.axis_index("c")
        s = lax.axis_index("s")

        pltpu.sync_copy(src_hbm.at[s], src_v)
        pltpu.sync_copy(dst_hbm.at[s], dst_v)
        if weighted:
            pltpu.sync_copy(ew_hbm.at[s], ew_v)

        def zero_zbuf(i, carry):
            for k in range(Q // L):
                zbuf[i, pl.ds(k * L, L)] = jnp.zeros((L,), jnp.float32)
            return carry
        lax.fori_loop(0, ZR, zero_zbuf, 0)

        def map_src(mul, delta):
            def arow(i, carry):
                for k in range(CHUNK // L):
                    sl = pl.ds(k * L, L)
                    src_v[i, sl] = src_v[i, sl] * mul + delta
                return carry
            lax.fori_loop(0, NCHUNK, arow, 0)

        def lane_bcast(v16, r):
            return lax.gather(
                v16, jnp.full((L, 1), r, jnp.int32),
                lax.GatherDimensionNumbers(
                    offset_dims=(), collapsed_slice_dims=(0,),
                    start_index_map=(0,)),
                slice_sizes=(1,),
                mode=lax.GatherScatterMode.PROMISE_IN_BOUNDS)

        GR = 4

        def scale(g, buf):
            @plsc.parallel_loop(0, CHUNK // L)
            def _(jj):
                ew16 = ew_v[g, pl.ds(jj * L, L)]
                for t in range(L // GR):
                    base = jj * L + t * GR
                    vals = [[buf[base + r, pl.ds(k * L, L)]
                             for k in range(Q // L)] for r in range(GR)]
                    ws = [lane_bcast(ew16, t * GR + r) for r in range(GR)]
                    for r in range(GR):
                        for k in range(Q // L):
                            buf[base + r, pl.ds(k * L, L)] = vals[r][k] * ws[r]

        def gissue(g, b):
            pltpu.async_copy(h_hbm.at[src_v.at[g]], rows[b], gsem[b])

        def gwait(g, b):
            pltpu.make_async_copy(h_hbm.at[src_v.at[g]], rows[b],
                                  gsem[b]).wait()

        def sissue(g, b):
            pltpu.async_copy(rows[b], acc_sh.at[dst_v.at[g]], ssem[b],
                             add=True)

        def swait(g, b):
            pltpu.make_async_copy(rows[b], acc_sh.at[dst_v.at[g]],
                                  ssem[b]).wait()

        def run_phase(qq):
            for k in range(RPW // ZR):
                pltpu.sync_copy(zbuf, acc_sh.at[pl.ds(s * RPW + k * ZR, ZR)])
            plsc.subcore_barrier()

            NR = NCHUNK // NB

            def step(g, b, gg, i):
                gwait(g, b)
                b3 = (b + 3) % NB
                if i < 2:
                    @pl.when(gg > 0)
                    def _():
                        swait(g - 2, b3)
                    gissue(g + 3, b3)
                else:
                    swait(g - 2, b3)
                    @pl.when(gg < NR - 1)
                    def _():
                        gissue(g + 3, b3)
                if weighted:
                    scale(g, rows[b])
                sissue(g, b)

            for i in range(3):
                gissue(i, i)

            def round_(gg, carry):
                g0 = gg * NB
                for i in range(NB):
                    step(g0 + i, i, gg, i)
                return carry
            lax.fori_loop(0, NR, round_, 0)

            for g in range(NCHUNK - 2, NCHUNK):
                swait(g, g % NB)

            plsc.subcore_barrier()
            pltpu.sync_copy(acc_sh.at[pl.ds(s * RPW, RPW)],
                            out_hbm.at[qq, pl.ds(s * RPW, RPW), c])

        map_src(2, c)
        run_phase(0)
        plsc.subcore_barrier()
        map_src(1, 2 * N)
        run_phase(1)

    return functools.partial(
        pl.kernel,
        out_type=jax.ShapeDtypeStruct((NC, NP, NC, Q), jnp.float32),
        mesh=mesh,
        scratch_types=scratch,
        compiler_params=pltpu.CompilerParams(use_tc_tiling_on_sc=False),
    )(body)


_sc_agg_w = _sc_agg(True)
_sc_agg_u = _sc_agg(False)

_BN = 1000


def _tc_first(x, W, b):
    def tc_body(x_ref, w_ref, b_ref, o_ref):
        h = jnp.dot(x_ref[...], w_ref[...],
                    preferred_element_type=jnp.float32) + b_ref[...]
        o_ref[0] = h[:, :D // 2]
        o_ref[1] = h[:, D // 2:]
    return pl.pallas_call(
        tc_body,
        grid=(N // _BN,),
        in_specs=[pl.BlockSpec((_BN, D), lambda i: (i, 0)),
                  pl.BlockSpec((D, D), lambda i: (0, 0)),
                  pl.BlockSpec((1, D), lambda i: (0, 0))],
        out_specs=pl.BlockSpec((NC, _BN, D // 2), lambda i: (0, i, 0)),
        out_shape=jax.ShapeDtypeStruct((NC, N, D // 2), jnp.float32),
    )(x, W, b.reshape(1, D))


def _tc_mid(a4, W, b):
    def tc_body(a_ref, w_ref, b_ref, o_ref):
        x0 = jnp.maximum(a_ref[0], 0.0)
        x1 = jnp.maximum(a_ref[1], 0.0)
        ss = (jnp.sum(x0 * x0, axis=1, keepdims=True)
              + jnp.sum(x1 * x1, axis=1, keepdims=True))
        scale = 1.0 / jnp.maximum(jnp.sqrt(ss), 1e-12)
        x = jnp.concatenate([x0, x1], axis=1) * scale
        h = jnp.dot(x, w_ref[...],
                    preferred_element_type=jnp.float32) + b_ref[...]
        o_ref[0] = h[:, :D // 2]
        o_ref[1] = h[:, D // 2:]
    return pl.pallas_call(
        tc_body,
        grid=(N // _BN,),
        in_specs=[pl.BlockSpec((NC, _BN, D // 2), lambda i: (0, i, 0)),
                  pl.BlockSpec((D, D), lambda i: (0, 0)),
                  pl.BlockSpec((1, D), lambda i: (0, 0))],
        out_specs=pl.BlockSpec((NC, _BN, D // 2), lambda i: (0, i, 0)),
        out_shape=jax.ShapeDtypeStruct((NC, N, D // 2), jnp.float32),
    )(a4, W, b.reshape(1, D))


def _tc_final(a2):
    def tc_body(a_ref, o_ref):
        o_ref[:, :D // 2] = a_ref[0]
        o_ref[:, D // 2:] = a_ref[1]
    return pl.pallas_call(
        tc_body,
        grid=(N // _BN,),
        in_specs=[pl.BlockSpec((NC, _BN, D // 2), lambda i: (0, i, 0))],
        out_specs=pl.BlockSpec((_BN, D), lambda i: (i, 0)),
        out_shape=jax.ShapeDtypeStruct((N, D), jnp.float32),
    )(a2)


def kernel(x, adj_t, edge_weight, W1, b1, W2, b2, W3, b3):
    src = adj_t[0].reshape(NS, NCHUNK, CHUNK).astype(jnp.int32)
    dst = adj_t[1].reshape(NS, NCHUNK, CHUNK).astype(jnp.int32)
    ew = edge_weight.reshape(NS, NCHUNK, CHUNK)

    h1 = _tc_first(x, W1, b1)
    a1 = _sc_agg_w(h1.reshape(NQ * N, Q), src, dst, ew)
    h2 = _tc_mid(a1.reshape(NC, N, D // 2), W2, b2)
    a2 = _sc_agg_w(h2.reshape(NQ * N, Q), src, dst, ew)
    h3 = _tc_mid(a2.reshape(NC, N, D // 2), W3, b3)
    a3 = _sc_agg_u(h3.reshape(NQ * N, Q), src, dst)
    return _tc_final(a3.reshape(NC, N, D // 2))


# speedup vs baseline: 1.1696x; 1.1696x over previous
import functools

import jax
import jax.numpy as jnp
from jax import lax
from jax.experimental import pallas as pl
from jax.experimental.pallas import tpu as pltpu
from jax.experimental.pallas import tpu_sc as plsc

N = 10000
E = 160000
D = 256
Q = 64
NQ = 4
NC = 2
NS = 16
L = 16
EPW = E // NS
CHUNK = 80
NCHUNK = EPW // CHUNK
NP = 10000
RPW = NP // NS
ZR = 125
NB = 5


def _sc_agg(weighted: bool):
    mesh = plsc.VectorSubcoreMesh(core_axis_name="c", subcore_axis_name="s",
                                  num_cores=NC, num_subcores=NS)

    scratch = [
        pltpu.VMEM((NCHUNK, CHUNK), jnp.int32),
        pltpu.VMEM((NCHUNK, CHUNK), jnp.int32),
        *[pltpu.VMEM((CHUNK, Q), jnp.float32) for _ in range(NB)],
        pltpu.VMEM((ZR, Q), jnp.float32),
        pltpu.VMEM_SHARED((NP, Q), jnp.float32),
        *[pltpu.SemaphoreType.DMA for _ in range(2 * NB)],
    ]
    if weighted:
        scratch.insert(2, pltpu.VMEM((NCHUNK, CHUNK), jnp.float32))

    def body(h_hbm, src_hbm, dst_hbm, *rest):
        if weighted:
            ew_hbm, out_hbm, src_v, dst_v, ew_v = rest[:5]
            rest = rest[5:]
        else:
            out_hbm, src_v, dst_v = rest[:3]
            rest = rest[3:]
        rows = rest[:NB]
        zbuf, acc_sh = rest[NB], rest[NB + 1]
        gsem = rest[NB + 2:NB + 2 + NB]
        ssem = rest[NB + 2 + NB:]
        c = lax.axis_index("c")
        s = lax.axis_index("s")

        pltpu.sync_copy(src_hbm.at[s], src_v)
        pltpu.sync_copy(dst_hbm.at[s], dst_v)
        if weighted:
            pltpu.sync_copy(ew_hbm.at[s], ew_v)

        def zero_zbuf(i, carry):
            for k in range(Q // L):
                zbuf[i, pl.ds(k * L, L)] = jnp.zeros((L,), jnp.float32)
            return carry
        lax.fori_loop(0, ZR, zero_zbuf, 0)

        def shift_src(delta):
            def arow(i, carry):
                for k in range(CHUNK // L):
                    sl = pl.ds(k * L, L)
                    src_v[i, sl] = src_v[i, sl] + delta
                return carry
            lax.fori_loop(0, NCHUNK, arow, 0)

        def lane_bcast(v16, r):
            return lax.gather(
                v16, jnp.full((L, 1), r, jnp.int32),
                lax.GatherDimensionNumbers(
                    offset_dims=(), collapsed_slice_dims=(0,),
                    start_index_map=(0,)),
                slice_sizes=(1,),
                mode=lax.GatherScatterMode.PROMISE_IN_BOUNDS)

        GR = 4

        def scale(g, buf):
            @plsc.parallel_loop(0, CHUNK // L)
            def _(jj):
                ew16 = ew_v[g, pl.ds(jj * L, L)]
                for t in range(L // GR):
                    base = jj * L + t * GR
                    vals = [[buf[base + r, pl.ds(k * L, L)]
                             for k in range(Q // L)] for r in range(GR)]
                    ws = [lane_bcast(ew16, t * GR + r) for r in range(GR)]
                    for r in range(GR):
                        for k in range(Q // L):
                            buf[base + r, pl.ds(k * L, L)] = vals[r][k] * ws[r]

        def gissue(g, b):
            pltpu.async_copy(h_hbm.at[src_v.at[g]], rows[b], gsem[b])

        def gwait(g, b):
            pltpu.make_async_copy(h_hbm.at[src_v.at[g]], rows[b],
                                  gsem[b]).wait()

        def sissue(g, b):
            pltpu.async_copy(rows[b], acc_sh.at[dst_v.at[g]], ssem[b],
                             add=True)

        def swait(g, b):
            pltpu.make_async_copy(rows[b], acc_sh.at[dst_v.at[g]],
                                  ssem[b]).wait()

        def run_phase(qq):
            for k in range(RPW // ZR):
                pltpu.sync_copy(zbuf, acc_sh.at[pl.ds(s * RPW + k * ZR, ZR)])
            plsc.subcore_barrier()

            NR = NCHUNK // NB

            def step(g, b, gg, i):
                gwait(g, b)
                b3 = (b + 3) % NB
                if i < 2:
                    @pl.when(gg > 0)
                    def _():
                        swait(g - 2, b3)
                    gissue(g + 3, b3)
                else:
                    swait(g - 2, b3)
                    @pl.when(gg < NR - 1)
                    def _():
                        gissue(g + 3, b3)
                if weighted:
                    scale(g, rows[b])
                sissue(g, b)

            for i in range(3):
                gissue(i, i)

            def round_(gg, carry):
                g0 = gg * NB
                for i in range(NB):
                    step(g0 + i, i, gg, i)
                return carry
            lax.fori_loop(0, NR, round_, 0)

            for g in range(NCHUNK - 2, NCHUNK):
                swait(g, g % NB)

            plsc.subcore_barrier()
            pltpu.sync_copy(acc_sh.at[pl.ds(s * RPW, RPW)],
                            out_hbm.at[pl.ds(qq * NP + s * RPW, RPW)])

        shift_src(c * N)
        run_phase(c)
        plsc.subcore_barrier()
        shift_src(2 * N)
        run_phase(2 + c)

    return functools.partial(
        pl.kernel,
        out_type=jax.ShapeDtypeStruct((NQ * NP, Q), jnp.float32),
        mesh=mesh,
        scratch_types=scratch,
        compiler_params=pltpu.CompilerParams(use_tc_tiling_on_sc=False),
    )(body)


_sc_agg_w = _sc_agg(True)
_sc_agg_u = _sc_agg(False)

_BN = 1000


def _tc_first(x, W, b):
    def tc_body(x_ref, w_ref, b_ref, o_ref):
        h = jnp.dot(x_ref[...], w_ref[...],
                    preferred_element_type=jnp.float32) + b_ref[...]
        for q in range(NQ):
            o_ref[q] = h[:, q * Q:(q + 1) * Q]
    return pl.pallas_call(
        tc_body,
        grid=(N // _BN,),
        in_specs=[pl.BlockSpec((_BN, D), lambda i: (i, 0)),
                  pl.BlockSpec((D, D), lambda i: (0, 0)),
                  pl.BlockSpec((1, D), lambda i: (0, 0))],
        out_specs=pl.BlockSpec((NQ, _BN, Q), lambda i: (0, i, 0)),
        out_shape=jax.ShapeDtypeStruct((NQ, N, Q), jnp.float32),
    )(x, W, b.reshape(1, D))


def _tc_mid(a4, W, b):
    def tc_body(a_ref, w_ref, b_ref, o_ref):
        xs = [jnp.maximum(a_ref[q], 0.0) for q in range(NQ)]
        ss = xs[0] * xs[0]
        for q in range(1, NQ):
            ss = ss + xs[q] * xs[q]
        ss = jnp.sum(ss, axis=1, keepdims=True)
        scale = 1.0 / jnp.maximum(jnp.sqrt(ss), 1e-12)
        x = jnp.concatenate(xs, axis=1) * scale
        h = jnp.dot(x, w_ref[...],
                    preferred_element_type=jnp.float32) + b_ref[...]
        for q in range(NQ):
            o_ref[q] = h[:, q * Q:(q + 1) * Q]
    return pl.pallas_call(
        tc_body,
        grid=(N // _BN,),
        in_specs=[pl.BlockSpec((NQ, _BN, Q), lambda i: (0, i, 0)),
                  pl.BlockSpec((D, D), lambda i: (0, 0)),
                  pl.BlockSpec((1, D), lambda i: (0, 0))],
        out_specs=pl.BlockSpec((NQ, _BN, Q), lambda i: (0, i, 0)),
        out_shape=jax.ShapeDtypeStruct((NQ, N, Q), jnp.float32),
    )(a4, W, b.reshape(1, D))


def kernel(x, adj_t, edge_weight, W1, b1, W2, b2, W3, b3):
    src = adj_t[0].reshape(NS, NCHUNK, CHUNK).astype(jnp.int32)
    dst = adj_t[1].reshape(NS, NCHUNK, CHUNK).astype(jnp.int32)
    ew = edge_weight.reshape(NS, NCHUNK, CHUNK)

    h1 = _tc_first(x, W1, b1)
    a1 = _sc_agg_w(h1.reshape(NQ * N, Q), src, dst, ew)
    h2 = _tc_mid(a1.reshape(NQ, N, Q), W2, b2)
    a2 = _sc_agg_w(h2.reshape(NQ * N, Q), src, dst, ew)
    h3 = _tc_mid(a2.reshape(NQ, N, Q), W3, b3)
    a3 = _sc_agg_u(h3.reshape(NQ * N, Q), src, dst)
    return a3.reshape(NQ, N, Q).transpose(1, 0, 2).reshape(N, D)
